# baseline (speedup 1.0000x reference)
import functools

import jax
import jax.numpy as jnp
import numpy as np
from jax import lax
from jax.experimental import pallas as pl
from jax.experimental.pallas import tpu as pltpu
from jax.experimental.pallas import tpu_sc as plsc

N = 10000
E = 320000
H = 128
L = 4
G = 500

NT = 16
NC = 2
C = 64
EP_PAD = 327680
EPT = EP_PAD // (NC * NT)
NCH = EPT // C
K = 32
NSUP = NCH // K
IROWS = EPT // C
NPD = 10240
RPT = NPD // NT
ZR = 64


def _edge_body(a0, a1, ep, src2, dst2, out0, out1,
               agg_s, sidx, didx, gb0, gb1, pb0, pb1,
               sem_i, sem_g0, sem_g1, sem_e0, sem_e1, sem_s0, sem_s1):
    c = lax.axis_index("c")
    s = lax.axis_index("s")
    t = c * NT + s
    gbufs = (gb0, gb1)
    pbufs = (pb0, pb1)
    gsems = (sem_g0, sem_g1)
    esems = (sem_e0, sem_e1)
    ssems = (sem_s0, sem_s1)

    @pl.loop(0, ZR)
    def _zero(i):
        for k in range(8):
            gb0[i, pl.ds(k * 16, 16)] = jnp.zeros((16,), jnp.float32)

    for i in range(RPT // ZR):
        pltpu.sync_copy(gb0, agg_s.at[pl.ds(s * RPT + i * ZR, ZR)])
    plsc.subcore_barrier()

    ebase = t * EPT

    def issue_loads(sup, i, b):
        @pl.when(c == 0)
        def _g0():
            pltpu.async_copy(a0.at[sidx.at[i]], gbufs[b], gsems[b])

        @pl.when(c == 1)
        def _g1():
            pltpu.async_copy(a1.at[sidx.at[i]], gbufs[b], gsems[b])

        pltpu.async_copy(ep.at[pl.ds(ebase + (sup * K + i) * C, C)],
                         pbufs[b], esems[b])

    def wait_loads(b):
        pltpu.make_async_copy(a0.at[sidx.at[0]], gbufs[b], gsems[b]).wait()
        pltpu.make_async_copy(ep.at[pl.ds(0, C)], pbufs[b], esems[b]).wait()

    def relu(b):
        gbuf, pbuf = gbufs[b], pbufs[b]

        @pl.loop(0, C)
        def _relu(j):
            for k in range(8):
                sl = pl.ds(k * 16, 16)
                pbuf[j, sl] = jnp.maximum(gbuf[j, sl] + pbuf[j, sl], 0.0)

    def issue_scatter(i, b):
        pltpu.async_copy(pbufs[b], agg_s.at[didx.at[i]], ssems[b], add=True)

    def wait_scatter(b):
        pltpu.make_async_copy(pbufs[b], agg_s.at[didx.at[0]], ssems[b]).wait()

    @pl.loop(0, NSUP)
    def _sup(sup):
        irow = t * IROWS + sup * K
        pltpu.sync_copy(src2.at[pl.ds(irow, K)], sidx)
        pltpu.sync_copy(dst2.at[pl.ds(irow, K)], didx)

        issue_loads(sup, 0, 0)
        issue_loads(sup, 1, 1)

        @pl.loop(0, K // 2 - 1)
        def _chunk(ii):
            i = ii * 2
            wait_loads(0)
            relu(0)
            issue_scatter(i, 0)
            wait_loads(1)
            relu(1)
            issue_scatter(i + 1, 1)
            wait_scatter(0)
            issue_loads(sup, i + 2, 0)
            wait_scatter(1)
            issue_loads(sup, i + 3, 1)

        wait_loads(0)
        relu(0)
        issue_scatter(K - 2, 0)
        wait_loads(1)
        relu(1)
        issue_scatter(K - 1, 1)
        wait_scatter(0)
        wait_scatter(1)

    plsc.subcore_barrier()

    @pl.when(c == 0)
    def _o0():
        pltpu.sync_copy(agg_s.at[pl.ds(s * RPT, RPT)],
                        out0.at[pl.ds(s * RPT, RPT)])

    @pl.when(c == 1)
    def _o1():
        pltpu.sync_copy(agg_s.at[pl.ds(s * RPT, RPT)],
                        out1.at[pl.ds(s * RPT, RPT)])


@jax.jit
def _edge_sc(a, ep, src2, dst2):
    a1 = a + 0.0
    mesh = plsc.VectorSubcoreMesh(core_axis_name="c", subcore_axis_name="s")
    fn = pl.kernel(
        _edge_body,
        out_type=(jax.ShapeDtypeStruct((NPD, H), jnp.float32),
                  jax.ShapeDtypeStruct((NPD, H), jnp.float32)),
        mesh=mesh,
        scratch_types=[
            pltpu.VMEM_SHARED((NPD, H), jnp.float32),
            pltpu.VMEM((K, C), jnp.int32),
            pltpu.VMEM((K, C), jnp.int32),
            pltpu.VMEM((C, H), jnp.float32),
            pltpu.VMEM((C, H), jnp.float32),
            pltpu.VMEM((C, H), jnp.float32),
            pltpu.VMEM((C, H), jnp.float32),
            pltpu.SemaphoreType.DMA,
            pltpu.SemaphoreType.DMA,
            pltpu.SemaphoreType.DMA,
            pltpu.SemaphoreType.DMA,
            pltpu.SemaphoreType.DMA,
            pltpu.SemaphoreType.DMA,
            pltpu.SemaphoreType.DMA,
        ],
    )
    return fn(a, a1, ep, src2, dst2)


def _out_proj_kernel(h_ref, w_ref, b_ref, o_ref):
    o_ref[...] = jax.nn.relu(
        jnp.dot(h_ref[...], w_ref[...], preferred_element_type=jnp.float32)
        + b_ref[...]
    )


def _out_proj(h, W_out, b_out):
    return pl.pallas_call(
        _out_proj_kernel,
        grid=(N // 400,),
        in_specs=[
            pl.BlockSpec((400, H), lambda i: (i, 0)),
            pl.BlockSpec((H, H), lambda i: (0, 0)),
            pl.BlockSpec((H,), lambda i: (0,)),
        ],
        out_specs=pl.BlockSpec((400, H), lambda i: (i, 0)),
        out_shape=jax.ShapeDtypeStruct((N, H), jnp.float32),
    )(h, W_out, b_out)


def kernel(x, z, edge_index, bond_feature, edge_attr, peripheral_attr, rd, pos,
           batch, z_table, W_init, b_init, W_msg, W_edge, W_self, ln_g, ln_b,
           Wv1, bv1, Wv2, bv2, W_out, b_out):
    z_emb = jnp.take(z_table, z, axis=0)
    h = jnp.concatenate([z_emb, x], axis=-1) @ W_init + b_init
    npad = EP_PAD - E
    src = edge_index[0].astype(jnp.int32)
    dst = edge_index[1].astype(jnp.int32)
    src2 = jnp.concatenate(
        [src, jnp.zeros((npad,), jnp.int32)]).reshape(EP_PAD // C, C)
    dst2 = jnp.concatenate(
        [dst, N + (jnp.arange(npad, dtype=jnp.int32) % (NPD - N))]
    ).reshape(EP_PAD // C, C)
    e = jnp.concatenate([bond_feature, edge_attr], axis=-1)
    e_pad = jnp.pad(e, ((0, npad), (0, 0)))
    vne = jnp.zeros((G, H), x.dtype)
    bn_scale = 1.0 / np.sqrt(1.0 + 1e-5)
    hcur = h
    for l in range(L):
        hl = hcur + jnp.take(vne, batch, axis=0)
        A = hl @ W_msg[l]
        Ep = e_pad @ W_edge[l]
        o0, o1 = _edge_sc(A, Ep, src2, dst2)
        agg = o0[:N] + o1[:N]
        hn = agg + hl @ W_self[l]
        mu = jnp.mean(hn, axis=-1, keepdims=True)
        var = jnp.var(hn, axis=-1, keepdims=True)
        hn = (hn - mu) / jnp.sqrt(var + 1e-5) * ln_g[l] + ln_b[l]
        if l < L - 1:
            tmp = jax.ops.segment_sum(hl, batch, num_segments=G) + vne
            t = jax.nn.relu((tmp @ Wv1[l] + bv1[l]) * bn_scale)
            t = jax.nn.relu((t @ Wv2[l] + bv2[l]) * bn_scale)
            vne = t
        hcur = hn
    return _out_proj(hcur, W_out, b_out)


# speedup vs baseline: 1.2251x; 1.2251x over previous
import functools

import jax
import jax.numpy as jnp
import numpy as np
from jax import lax
from jax.experimental import pallas as pl
from jax.experimental.pallas import tpu as pltpu
from jax.experimental.pallas import tpu_sc as plsc

N = 10000
E = 320000
H = 128
L = 4
G = 500

NT = 16
NC = 2
C = 64
EP_PAD = 327680
EPT0 = 14336
EPT1 = 6144
K = 32
NPD = 10240
RPT = NPD // NT
ZR = 64


def _edge_body(a, ep, src2, dst2, out0, out1,
               agg_s, sidx, didx, gb0, gb1, pb0, pb1,
               sem_i, sem_g0, sem_g1, sem_e0, sem_e1, sem_s0, sem_s1):
    c = lax.axis_index("c")
    s = lax.axis_index("s")
    t = c * NT + s
    gbufs = (gb0, gb1)
    pbufs = (pb0, pb1)
    gsems = (sem_g0, sem_g1)
    esems = (sem_e0, sem_e1)
    ssems = (sem_s0, sem_s1)

    @pl.loop(0, ZR)
    def _zero(i):
        for k in range(8):
            gb0[i, pl.ds(k * 16, 16)] = jnp.zeros((16,), jnp.float32)

    for i in range(RPT // ZR):
        pltpu.sync_copy(gb0, agg_s.at[pl.ds(s * RPT + i * ZR, ZR)])
    plsc.subcore_barrier()

    ebase0 = s * EPT0
    ebase1 = NT * EPT0 + s * EPT1
    ebase = pl.multiple_of(jnp.where(c == 0, ebase0, ebase1), 512)
    nsup = jnp.where(c == 0, EPT0 // (K * C), EPT1 // (K * C))

    def issue_loads(sup, i, b):
        pltpu.async_copy(a.at[sidx.at[i]], gbufs[b], gsems[b])
        pltpu.async_copy(ep.at[pl.ds(ebase + (sup * K + i) * C, C)],
                         pbufs[b], esems[b])

    def wait_loads(b):
        pltpu.make_async_copy(a.at[sidx.at[0]], gbufs[b], gsems[b]).wait()
        pltpu.make_async_copy(ep.at[pl.ds(0, C)], pbufs[b], esems[b]).wait()

    def relu(b):
        gbuf, pbuf = gbufs[b], pbufs[b]

        @pl.loop(0, C)
        def _relu(j):
            for k in range(8):
                sl = pl.ds(k * 16, 16)
                pbuf[j, sl] = jnp.maximum(gbuf[j, sl] + pbuf[j, sl], 0.0)

    def issue_scatter(i, b):
        pltpu.async_copy(pbufs[b], agg_s.at[didx.at[i]], ssems[b], add=True)

    def wait_scatter(b):
        pltpu.make_async_copy(pbufs[b], agg_s.at[didx.at[0]], ssems[b]).wait()

    @pl.loop(0, nsup)
    def _sup(sup):
        irow = pl.multiple_of(ebase // C + sup * K, 8)
        pltpu.sync_copy(src2.at[pl.ds(irow, K)], sidx)
        pltpu.sync_copy(dst2.at[pl.ds(irow, K)], didx)

        issue_loads(sup, 0, 0)
        issue_loads(sup, 1, 1)

        @pl.loop(0, K // 2 - 1)
        def _chunk(ii):
            i = ii * 2
            wait_loads(0)
            relu(0)
            issue_scatter(i, 0)
            wait_loads(1)
            relu(1)
            issue_scatter(i + 1, 1)
            wait_scatter(0)
            issue_loads(sup, i + 2, 0)
            wait_scatter(1)
            issue_loads(sup, i + 3, 1)

        wait_loads(0)
        relu(0)
        issue_scatter(K - 2, 0)
        wait_loads(1)
        relu(1)
        issue_scatter(K - 1, 1)
        wait_scatter(0)
        wait_scatter(1)

    plsc.subcore_barrier()

    @pl.when(c == 0)
    def _o0():
        pltpu.sync_copy(agg_s.at[pl.ds(s * RPT, RPT)],
                        out0.at[pl.ds(s * RPT, RPT)])

    @pl.when(c == 1)
    def _o1():
        pltpu.sync_copy(agg_s.at[pl.ds(s * RPT, RPT)],
                        out1.at[pl.ds(s * RPT, RPT)])


@jax.jit
def _edge_sc(a, ep, src2, dst2):
    mesh = plsc.VectorSubcoreMesh(core_axis_name="c", subcore_axis_name="s")
    fn = pl.kernel(
        _edge_body,
        out_type=(jax.ShapeDtypeStruct((NPD, H), jnp.float32),
                  jax.ShapeDtypeStruct((NPD, H), jnp.float32)),
        mesh=mesh,
        scratch_types=[
            pltpu.VMEM_SHARED((NPD, H), jnp.float32),
            pltpu.VMEM((K, C), jnp.int32),
            pltpu.VMEM((K, C), jnp.int32),
            pltpu.VMEM((C, H), jnp.float32),
            pltpu.VMEM((C, H), jnp.float32),
            pltpu.VMEM((C, H), jnp.float32),
            pltpu.VMEM((C, H), jnp.float32),
            pltpu.SemaphoreType.DMA,
            pltpu.SemaphoreType.DMA,
            pltpu.SemaphoreType.DMA,
            pltpu.SemaphoreType.DMA,
            pltpu.SemaphoreType.DMA,
            pltpu.SemaphoreType.DMA,
            pltpu.SemaphoreType.DMA,
        ],
    )
    return fn(a, ep, src2, dst2)


def _out_proj_kernel(h_ref, w_ref, b_ref, o_ref):
    o_ref[...] = jax.nn.relu(
        jnp.dot(h_ref[...], w_ref[...], preferred_element_type=jnp.float32)
        + b_ref[...]
    )


def _out_proj(h, W_out, b_out):
    return pl.pallas_call(
        _out_proj_kernel,
        grid=(N // 400,),
        in_specs=[
            pl.BlockSpec((400, H), lambda i: (i, 0)),
            pl.BlockSpec((H, H), lambda i: (0, 0)),
            pl.BlockSpec((H,), lambda i: (0,)),
        ],
        out_specs=pl.BlockSpec((400, H), lambda i: (i, 0)),
        out_shape=jax.ShapeDtypeStruct((N, H), jnp.float32),
    )(h, W_out, b_out)


def kernel(x, z, edge_index, bond_feature, edge_attr, peripheral_attr, rd, pos,
           batch, z_table, W_init, b_init, W_msg, W_edge, W_self, ln_g, ln_b,
           Wv1, bv1, Wv2, bv2, W_out, b_out):
    z_emb = jnp.take(z_table, z, axis=0)
    h = jnp.concatenate([z_emb, x], axis=-1) @ W_init + b_init
    npad = EP_PAD - E
    src = edge_index[0].astype(jnp.int32)
    dst = edge_index[1].astype(jnp.int32)
    src2 = jnp.concatenate(
        [src, jnp.zeros((npad,), jnp.int32)]).reshape(EP_PAD // C, C)
    dst2 = jnp.concatenate(
        [dst, N + (jnp.arange(npad, dtype=jnp.int32) % (NPD - N))]
    ).reshape(EP_PAD // C, C)
    e = jnp.concatenate([bond_feature, edge_attr], axis=-1)
    e_pad = jnp.pad(e, ((0, npad), (0, 0)))
    vne = jnp.zeros((G, H), x.dtype)
    bn_scale = 1.0 / np.sqrt(1.0 + 1e-5)
    hcur = h
    for l in range(L):
        hl = hcur + jnp.take(vne, batch, axis=0)
        A = hl @ W_msg[l]
        Ep = e_pad @ W_edge[l]
        o0, o1 = _edge_sc(A, Ep, src2, dst2)
        agg = o0[:N] + o1[:N]
        hn = agg + hl @ W_self[l]
        mu = jnp.mean(hn, axis=-1, keepdims=True)
        var = jnp.var(hn, axis=-1, keepdims=True)
        hn = (hn - mu) / jnp.sqrt(var + 1e-5) * ln_g[l] + ln_b[l]
        if l < L - 1:
            tmp = jax.ops.segment_sum(hl, batch, num_segments=G) + vne
            t = jax.nn.relu((tmp @ Wv1[l] + bv1[l]) * bn_scale)
            t = jax.nn.relu((t @ Wv2[l] + bv2[l]) * bn_scale)
            vne = t
        hcur = hn
    return _out_proj(hcur, W_out, b_out)
